# baseline (speedup 1.0000x reference)
import functools

import jax
import jax.numpy as jnp
from jax import lax
from jax.experimental import pallas as pl
from jax.experimental.pallas import tpu as pltpu
from jax.experimental.pallas import tpu_sc as plsc

VOCAB = 1000
D = 1000
BATCH = 1024
SEQ = 50
NC = 2
NS = 16
NW = NC * NS

_mesh = plsc.VectorSubcoreMesh(
    core_axis_name="c", subcore_axis_name="s", num_cores=NC, num_subcores=NS
)

_scratch = (
    [pltpu.VMEM((SEQ, 128), jnp.int32)]
    + [pltpu.VMEM((128,), jnp.int32) for _ in range(2)]
    + [pltpu.VMEM((128, 128), jnp.float32) for _ in range(4)]
    + [pltpu.SemaphoreType.DMA for _ in range(4)]
)


@functools.partial(
    pl.kernel,
    mesh=_mesh,
    out_type=jax.ShapeDtypeStruct((SEQ, D, BATCH), jnp.float32),
    scratch_types=_scratch,
    compiler_params=pltpu.CompilerParams(needs_layout_passes=False),
)
def _emb_lookup(
    xt_hbm, table_hbm, out_hbm,
    idx_v, ipv0, ipv1, buf0, buf1, tb0, tb1, gs0, gs1, ss0, ss1,
):
    ipvs = (ipv0, ipv1)
    bufs = (buf0, buf1)
    tbs = (tb0, tb1)
    gsems = (gs0, gs1)
    ssems = (ss0, ss1)

    wid = lax.axis_index("s") * NC + lax.axis_index("c")
    bq = wid // 4
    j = wid % 4

    pltpu.sync_copy(xt_hbm.at[:, pl.ds(bq * 128, 128)], idx_v)

    def ipv_compute(s, u):
        dc = 2 * j + u
        for k in range(8):
            v = idx_v[s, pl.ds(16 * k, 16)]
            ipvs[u][pl.ds(16 * k, 16)] = v * 8 + dc

    def gdesc(u):
        return pltpu.make_async_copy(table_hbm.at[ipvs[u]], bufs[u], gsems[u])

    def sdesc_full(s, u):
        dc = 2 * j + u
        return pltpu.make_async_copy(
            tbs[u],
            out_hbm.at[s, pl.ds(dc * 128, 128), pl.ds(bq * 128, 128)],
            ssems[u],
        )

    def sdesc_part(s):
        return pltpu.make_async_copy(
            tbs[1].at[pl.ds(0, D - 896)],
            out_hbm.at[s, pl.ds(896, D - 896), pl.ds(bq * 128, 128)],
            ssems[1],
        )

    def sstart(s, u):
        if u == 0:
            sdesc_full(s, u).start()
        else:
            @pl.when(j == 3)
            def _():
                sdesc_part(s).start()

            @pl.when(j != 3)
            def _():
                sdesc_full(s, u).start()

    def swait(s, u):
        if u == 0:
            sdesc_full(s, u).wait()
        else:
            @pl.when(j == 3)
            def _():
                sdesc_part(s).wait()

            @pl.when(j != 3)
            def _():
                sdesc_full(s, u).wait()

    def transpose(u):
        def dbody(c, carry):
            cols = jnp.full((16,), c, jnp.int32)
            for k in range(8):
                rows = lax.iota(jnp.int32, 16) + 16 * k
                tbs[u][c, pl.ds(16 * k, 16)] = plsc.load_gather(
                    bufs[u], [rows, cols]
                )
            return carry

        lax.fori_loop(0, 128, dbody, 0)

    def work(s, first, last):
        for u in (0, 1):
            gdesc(u).wait()
            if not first:
                swait(s - 1, u)
            transpose(u)
            sstart(s, u)
            if not last:
                ipv_compute(s + 1, u)
                gdesc(u).start()

    for u in (0, 1):
        ipv_compute(0, u)
        gdesc(u).start()
    work(0, True, False)
    lax.fori_loop(1, SEQ - 1, lambda s, c: (work(s, False, False), c)[1], 0)
    work(SEQ - 1, False, True)
    for u in (0, 1):
        swait(SEQ - 1, u)


def kernel(x, embedding):
    xt = x.T.astype(jnp.int32)
    tab2 = jnp.pad(embedding, ((0, 0), (0, 1024 - D))).reshape(8 * VOCAB, 128)
    out3 = _emb_lookup(xt, tab2)
    return jnp.transpose(out3, (2, 0, 1))


# speedup vs baseline: 1.8655x; 1.8655x over previous
import functools

import jax
import jax.numpy as jnp
from jax import lax
from jax.experimental import pallas as pl
from jax.experimental.pallas import tpu as pltpu
from jax.experimental.pallas import tpu_sc as plsc

VOCAB = 1000
D = 1000
BATCH = 1024
SEQ = 50
NC = 2
NS = 16
NW = NC * NS

_mesh = plsc.VectorSubcoreMesh(
    core_axis_name="c", subcore_axis_name="s", num_cores=NC, num_subcores=NS
)

_scratch = (
    [pltpu.VMEM((SEQ, 128), jnp.int32)]
    + [pltpu.VMEM((128,), jnp.int32) for _ in range(2)]
    + [pltpu.VMEM((128, 128), jnp.float32) for _ in range(4)]
    + [pltpu.SemaphoreType.DMA for _ in range(4)]
)


@functools.partial(
    pl.kernel,
    mesh=_mesh,
    out_type=jax.ShapeDtypeStruct((SEQ, D, BATCH), jnp.float32),
    scratch_types=_scratch,
    compiler_params=pltpu.CompilerParams(needs_layout_passes=False),
)
def _emb_lookup(
    xt_hbm, table_hbm, out_hbm,
    idx_v, ipv0, ipv1, buf0, buf1, tb0, tb1, gs0, gs1, ss0, ss1,
):
    ipvs = (ipv0, ipv1)
    bufs = (buf0, buf1)
    tbs = (tb0, tb1)
    gsems = (gs0, gs1)
    ssems = (ss0, ss1)

    wid = lax.axis_index("s") * NC + lax.axis_index("c")
    bq = wid // 4
    j = wid % 4

    pltpu.sync_copy(xt_hbm.at[:, pl.ds(bq * 128, 128)], idx_v)

    def ipv_compute(s, u):
        dc = 2 * j + u
        for k in range(8):
            v = idx_v[s, pl.ds(16 * k, 16)]
            ipvs[u][pl.ds(16 * k, 16)] = v * 8 + dc

    def gdesc(u):
        return pltpu.make_async_copy(table_hbm.at[ipvs[u]], bufs[u], gsems[u])

    def sdesc_full(s, u):
        dc = 2 * j + u
        return pltpu.make_async_copy(
            tbs[u],
            out_hbm.at[s, pl.ds(dc * 128, 128), pl.ds(bq * 128, 128)],
            ssems[u],
        )

    def sdesc_part(s):
        return pltpu.make_async_copy(
            tbs[1].at[pl.ds(0, D - 896)],
            out_hbm.at[s, pl.ds(896, D - 896), pl.ds(bq * 128, 128)],
            ssems[1],
        )

    def sstart(s, u):
        if u == 0:
            sdesc_full(s, u).start()
        else:
            @pl.when(j == 3)
            def _():
                sdesc_part(s).start()

            @pl.when(j != 3)
            def _():
                sdesc_full(s, u).start()

    def swait(s, u):
        if u == 0:
            sdesc_full(s, u).wait()
        else:
            @pl.when(j == 3)
            def _():
                sdesc_part(s).wait()

            @pl.when(j != 3)
            def _():
                sdesc_full(s, u).wait()

    def transpose(u):
        @plsc.parallel_loop(0, 128, unroll=8)
        def dbody(c):
            cols = jnp.full((16,), c, jnp.int32)
            for k in range(8):
                rows = lax.iota(jnp.int32, 16) + 16 * k
                tbs[u][c, pl.ds(16 * k, 16)] = plsc.load_gather(
                    bufs[u], [rows, cols]
                )

    def work(s, first, last):
        for u in (0, 1):
            gdesc(u).wait()
            if not first:
                swait(s - 1, u)
            transpose(u)
            sstart(s, u)
            if not last:
                ipv_compute(s + 1, u)
                gdesc(u).start()

    for u in (0, 1):
        ipv_compute(0, u)
        gdesc(u).start()
    work(0, True, False)
    lax.fori_loop(1, SEQ - 1, lambda s, c: (work(s, False, False), c)[1], 0)
    work(SEQ - 1, False, True)
    for u in (0, 1):
        swait(SEQ - 1, u)


def kernel(x, embedding):
    xt = x.T.astype(jnp.int32)
    tab2 = jnp.pad(embedding, ((0, 0), (0, 1024 - D))).reshape(8 * VOCAB, 128)
    out3 = _emb_lookup(xt, tab2)
    return jnp.transpose(out3, (2, 0, 1))
